# baseline (speedup 1.0000x reference)
import functools

import jax
import jax.numpy as jnp
from jax import lax
from jax.experimental import pallas as pl
from jax.experimental.pallas import tpu as pltpu
from jax.experimental.pallas import tpu_sc as plsc

_NC = 2
_NS = 16
_NW = _NC * _NS


def _sc_mesh():
    return plsc.VectorSubcoreMesh(core_axis_name="c", subcore_axis_name="s",
                                  num_cores=_NC, num_subcores=_NS)


def _reduce_groups(rows, CHN, DEG, NV, outv, out_base, D):
    zero = jnp.zeros((16,), jnp.float32)

    def jbody(j, accs):
        return tuple(
            accs[g * NV + v] + rows[g * DEG + j, pl.ds(v * 16, 16)]
            for g in range(CHN) for v in range(NV))

    accs = lax.fori_loop(0, DEG, jbody, (zero,) * (CHN * NV))
    for g in range(CHN):
        for v in range(NV):
            outv[pl.ds(out_base + g * D + v * 16, 16)] = accs[g * NV + v]


def _make_agg_all(N, D, DEG, NP):
    CHI = 128
    CHN = CHI // DEG
    R = NP // _NW
    NCH = (R * DEG) // CHI
    NV = D // 16

    NBUF = 2

    RS = -(-(N // _NS) // 8) * 8
    RS_LAST = N - (_NS - 1) * RS

    def body(adj_hbm, feat_hbm, out_hbm, adjv, rows0, rows1, oc0, oc1,
             shared, sem0, sem1, semo0, semo1):
        c = lax.axis_index("c")
        s = lax.axis_index("s")
        wid = s * _NC + c
        bufs = ((rows0, sem0, oc0, semo0), (rows1, sem1, oc1, semo1))

        @pl.when(s < _NS - 1)
        def _():
            pltpu.sync_copy(feat_hbm.at[pl.ds(s * RS, RS)],
                            shared.at[pl.ds(s * RS, RS)])

        @pl.when(s == _NS - 1)
        def _():
            pltpu.sync_copy(feat_hbm.at[pl.ds((_NS - 1) * RS, RS_LAST)],
                            shared.at[pl.ds((_NS - 1) * RS, RS_LAST)])
        pltpu.sync_copy(adj_hbm.at[wid], adjv)
        plsc.subcore_barrier()

        def out_slice(cc):
            return out_hbm.at[pl.ds((wid * R + cc * CHN) * D, CHN * D)]

        for b, (rows, sem, _oc, _semo) in enumerate(bufs):
            pltpu.async_copy(shared.at[adjv.at[b]], rows, sem)

        def chunk_group(i, _):
            for b, (rows, sem, oc, semo) in enumerate(bufs):
                cc = i * NBUF + b
                pltpu.make_async_copy(
                    shared.at[adjv.at[cc]], rows, sem).wait()
                @pl.when(cc >= NBUF)
                def _():
                    pltpu.make_async_copy(oc, out_slice(cc - NBUF), semo).wait()
                _reduce_groups(rows, CHN, DEG, NV, oc, 0, D)
                pltpu.async_copy(oc, out_slice(cc), semo)
                nxt = cc + NBUF
                hbm_turn = (nxt & 7) == 7

                @pl.when((nxt < NCH) & hbm_turn)
                def _():
                    pltpu.async_copy(feat_hbm.at[adjv.at[nxt]], rows, sem)

                @pl.when((nxt < NCH) & jnp.logical_not(hbm_turn))
                def _():
                    pltpu.async_copy(shared.at[adjv.at[nxt]], rows, sem)
            return 0

        lax.fori_loop(0, NCH // NBUF, chunk_group, 0)
        for b, (rows, sem, oc, semo) in enumerate(bufs):
            pltpu.make_async_copy(oc, out_slice(NCH - NBUF + b), semo).wait()

    kern = pl.kernel(
        body,
        out_type=jax.ShapeDtypeStruct((NP * D,), jnp.float32),
        mesh=_sc_mesh(),
        scratch_types=[
            pltpu.VMEM((NCH, CHI), jnp.int32),
            pltpu.VMEM((CHI, D), jnp.float32),
            pltpu.VMEM((CHI, D), jnp.float32),
            pltpu.VMEM((CHN * D,), jnp.float32),
            pltpu.VMEM((CHN * D,), jnp.float32),
            pltpu.VMEM_SHARED((N, D), jnp.float32),
            pltpu.SemaphoreType.DMA,
            pltpu.SemaphoreType.DMA,
            pltpu.SemaphoreType.DMA,
            pltpu.SemaphoreType.DMA,
        ],
    )
    return kern


def _make_seed_agg(N, H, DEG, B):
    SB = B // _NW
    NV = H // 16
    CHI = 128
    CHN = CHI // DEG
    NCH = (SB * DEG) // CHI

    NBUF = 2

    def body(nodes_hbm, adj_hbm, h1_hbm, self_hbm, agg_hbm,
             nodesv, qrow, adjrows, idxv, selfv, neigh0, neigh1, aggv,
             sem0, sem1, sem_self):
        c = lax.axis_index("c")
        s = lax.axis_index("s")
        wid = s * _NC + c
        base = wid * SB
        bufs = ((neigh0, sem0), (neigh1, sem1))[:NBUF]

        pltpu.sync_copy(nodes_hbm.at[pl.ds(base, SB)], nodesv)
        QR = 128 // DEG
        qsh = QR.bit_length() - 1
        for k in range(SB // 16):
            qrow[pl.ds(k * 16, 16)] = nodesv[pl.ds(k * 16, 16)] >> qsh
        pltpu.async_copy(adj_hbm.at[qrow], adjrows, sem0).wait()
        cp_self = pltpu.async_copy(h1_hbm.at[nodesv], selfv, sem_self)

        for blk16 in range(SB // 16):
            qvec = (nodesv[pl.ds(blk16 * 16, 16)] & (QR - 1)) * DEG
            for m in range(16):
                si = blk16 * 16 + m
                start = qvec[m]
                for k in range(DEG // 16):
                    flat = si * DEG + k * 16
                    idxv[flat // CHI, pl.ds(flat % CHI, 16)] = (
                        adjrows[si, pl.ds(start + k * 16, 16)])

        for b, (neigh, sem) in enumerate(bufs):
            pltpu.async_copy(h1_hbm.at[idxv.at[b]], neigh, sem)

        def chunk_group(i, _):
            for b, (neigh, sem) in enumerate(bufs):
                cc = i * NBUF + b
                pltpu.make_async_copy(
                    h1_hbm.at[idxv.at[cc]], neigh, sem).wait()
                _reduce_groups(neigh, CHN, DEG, NV, aggv, cc * CHN * H, H)
                @pl.when(cc + NBUF < NCH)
                def _():
                    pltpu.async_copy(
                        h1_hbm.at[idxv.at[cc + NBUF]], neigh, sem)
            return 0

        lax.fori_loop(0, NCH // NBUF, chunk_group, 0)
        cp_self.wait()
        pltpu.sync_copy(selfv, self_hbm.at[pl.ds(base, SB)])
        pltpu.sync_copy(aggv, agg_hbm.at[pl.ds(base * H, SB * H)])

    kern = pl.kernel(
        body,
        out_type=(
            jax.ShapeDtypeStruct((B, H), jnp.float32),
            jax.ShapeDtypeStruct((B * H,), jnp.float32),
        ),
        mesh=_sc_mesh(),
        scratch_types=[
            pltpu.VMEM((SB,), jnp.int32),
            pltpu.VMEM((SB,), jnp.int32),
            pltpu.VMEM((SB, 128), jnp.int32),
            pltpu.VMEM((NCH, CHI), jnp.int32),
            pltpu.VMEM((SB, H), jnp.float32),
            pltpu.VMEM((CHI, H), jnp.float32),
            pltpu.VMEM((CHI, H), jnp.float32),
            pltpu.VMEM((SB * H,), jnp.float32),
            pltpu.SemaphoreType.DMA,
            pltpu.SemaphoreType.DMA,
            pltpu.SemaphoreType.DMA,
        ],
    )
    return kern


def _layer1_matmul(x, agg, w1aT, w1bT, DEG, blk):
    M, D = x.shape
    H = w1aT.shape[1]
    inv = 1.0 / DEG

    def body(x_ref, g_ref, wa_ref, wb_ref, o_ref):
        h = jnp.dot(x_ref[...], wa_ref[...], preferred_element_type=jnp.float32)
        h = h + jnp.dot(g_ref[...] * inv, wb_ref[...],
                        preferred_element_type=jnp.float32)
        o_ref[...] = jnp.maximum(h, 0.0)

    return pl.pallas_call(
        body,
        grid=(M // blk,),
        in_specs=[
            pl.BlockSpec((blk, D), lambda i: (i, 0)),
            pl.BlockSpec((blk, D), lambda i: (i, 0)),
            pl.BlockSpec((D, H), lambda i: (0, 0)),
            pl.BlockSpec((D, H), lambda i: (0, 0)),
        ],
        out_specs=pl.BlockSpec((blk, H), lambda i: (i, 0)),
        out_shape=jax.ShapeDtypeStruct((M, H), jnp.float32),
    )(x, agg, w1aT, w1bT)


def _layer2_matmul(self_emb, agg2, w2aT, w2bT, cw, DEG):
    B, H = self_emb.shape
    OUT = cw.shape[1]
    inv = 1.0 / DEG

    def body(s_ref, g_ref, wa_ref, wb_ref, cw_ref, o_ref):
        h = jnp.dot(s_ref[...], wa_ref[...], preferred_element_type=jnp.float32)
        h = h + jnp.dot(g_ref[...] * inv, wb_ref[...],
                        preferred_element_type=jnp.float32)
        h = jnp.maximum(h, 0.0)
        o_ref[...] = jnp.dot(h, cw_ref[...], preferred_element_type=jnp.float32)

    return pl.pallas_call(
        body,
        out_shape=jax.ShapeDtypeStruct((B, OUT), jnp.float32),
    )(self_emb, agg2, w2aT, w2bT, cw)


def kernel(nodes, adj, features, w1, w2, cw):
    N, D = features.shape
    DEG = adj.shape[1]
    B = nodes.shape[0]
    H = w1.shape[0]

    CHN = 128 // DEG
    step = _NW * CHN * 4
    NP = ((N + step - 1) // step) * step
    adj_p = jnp.pad(adj, ((0, NP - N), (0, 0)))
    adj3 = adj_p.reshape(_NW, (NP // _NW * DEG) // 128, 128)

    agg_flat = _make_agg_all(N, D, DEG, NP)(adj3, features)
    agg1 = agg_flat.reshape(NP, D)

    w1aT = w1[:, :D].T
    w1bT = w1[:, D:].T
    h1 = _layer1_matmul(features, agg1, w1aT, w1bT, DEG, blk=N // 10)

    adj4 = adj_p.reshape((NP * DEG) // 128, 128)
    self_emb, agg2_flat = _make_seed_agg(N, H, DEG, B)(nodes, adj4, h1)
    agg2 = agg2_flat.reshape(B, H)

    w2aT = w2[:, :H].T
    w2bT = w2[:, H:].T
    return _layer2_matmul(self_emb, agg2, w2aT, w2bT, cw, DEG)


# speedup vs baseline: 1.3928x; 1.3928x over previous
import jax
import jax.numpy as jnp
from jax import lax
from jax.experimental import pallas as pl
from jax.experimental.pallas import tpu as pltpu
from jax.experimental.pallas import tpu_sc as plsc

_NC = 2
_NS = 16
_NW = _NC * _NS


def _sc_mesh():
    return plsc.VectorSubcoreMesh(core_axis_name="c", subcore_axis_name="s",
                                  num_cores=_NC, num_subcores=_NS)


def _reduce_groups(rows, CHN, DEG, NV, outv, out_base, D):
    zero = jnp.zeros((16,), jnp.float32)

    def jbody(j, accs):
        return tuple(
            accs[g * NV + v] + rows[g * DEG + j, pl.ds(v * 16, 16)]
            for g in range(CHN) for v in range(NV))

    accs = lax.fori_loop(0, DEG, jbody, (zero,) * (CHN * NV))
    for g in range(CHN):
        for v in range(NV):
            outv[pl.ds(out_base + g * D + v * 16, 16)] = accs[g * NV + v]


def _make_agg_all(N, D, DEG, NP):
    CHI = 128
    CHN = CHI // DEG
    R = NP // _NW
    NCH = (R * DEG) // CHI
    NV = D // 16

    NBUF = 2

    RS = -(-(N // _NS) // 8) * 8
    RS_LAST = N - (_NS - 1) * RS

    def body(adj_hbm, feat_hbm, out_hbm, adjv, rows0, rows1, oc0, oc1,
             shared, sem0, sem1, semo0, semo1):
        c = lax.axis_index("c")
        s = lax.axis_index("s")
        wid = s * _NC + c
        bufs = ((rows0, sem0, oc0, semo0), (rows1, sem1, oc1, semo1))

        @pl.when(s < _NS - 1)
        def _():
            pltpu.sync_copy(feat_hbm.at[pl.ds(s * RS, RS)],
                            shared.at[pl.ds(s * RS, RS)])

        @pl.when(s == _NS - 1)
        def _():
            pltpu.sync_copy(feat_hbm.at[pl.ds((_NS - 1) * RS, RS_LAST)],
                            shared.at[pl.ds((_NS - 1) * RS, RS_LAST)])
        pltpu.sync_copy(adj_hbm.at[wid], adjv)
        plsc.subcore_barrier()

        def out_slice(cc):
            return out_hbm.at[pl.ds((wid * R + cc * CHN) * D, CHN * D)]

        for b, (rows, sem, _oc, _semo) in enumerate(bufs):
            pltpu.async_copy(shared.at[adjv.at[b]], rows, sem)

        def chunk_group(i, _):
            for b, (rows, sem, oc, semo) in enumerate(bufs):
                cc = i * NBUF + b
                pltpu.make_async_copy(
                    shared.at[adjv.at[cc]], rows, sem).wait()
                @pl.when(cc >= NBUF)
                def _():
                    pltpu.make_async_copy(oc, out_slice(cc - NBUF), semo).wait()
                _reduce_groups(rows, CHN, DEG, NV, oc, 0, D)
                pltpu.async_copy(oc, out_slice(cc), semo)
                @pl.when(cc + NBUF < NCH)
                def _():
                    pltpu.async_copy(
                        shared.at[adjv.at[cc + NBUF]], rows, sem)
            return 0

        lax.fori_loop(0, NCH // NBUF, chunk_group, 0)
        for b, (rows, sem, oc, semo) in enumerate(bufs):
            pltpu.make_async_copy(oc, out_slice(NCH - NBUF + b), semo).wait()

    kern = pl.kernel(
        body,
        out_type=jax.ShapeDtypeStruct((NP * D,), jnp.float32),
        mesh=_sc_mesh(),
        scratch_types=[
            pltpu.VMEM((NCH, CHI), jnp.int32),
            pltpu.VMEM((CHI, D), jnp.float32),
            pltpu.VMEM((CHI, D), jnp.float32),
            pltpu.VMEM((CHN * D,), jnp.float32),
            pltpu.VMEM((CHN * D,), jnp.float32),
            pltpu.VMEM_SHARED((N, D), jnp.float32),
            pltpu.SemaphoreType.DMA,
            pltpu.SemaphoreType.DMA,
            pltpu.SemaphoreType.DMA,
            pltpu.SemaphoreType.DMA,
        ],
    )
    return kern


def _make_seed_agg(N, H, DEG, B):
    SB = B // _NW
    NV = H // 16
    CHI = 128
    CHN = CHI // DEG
    NCH = (SB * DEG) // CHI

    NBUF = 2

    def body(nodes_hbm, adj_hbm, h1_hbm, self_hbm, agg_hbm,
             nodesv, qrow, adjrows, idxv, selfv, neigh0, neigh1, aggv,
             sem0, sem1, sem_self):
        c = lax.axis_index("c")
        s = lax.axis_index("s")
        wid = s * _NC + c
        base = wid * SB
        bufs = ((neigh0, sem0), (neigh1, sem1))[:NBUF]

        pltpu.sync_copy(nodes_hbm.at[pl.ds(base, SB)], nodesv)
        QR = 128 // DEG
        qsh = QR.bit_length() - 1
        for k in range(SB // 16):
            qrow[pl.ds(k * 16, 16)] = nodesv[pl.ds(k * 16, 16)] >> qsh
        pltpu.async_copy(adj_hbm.at[qrow], adjrows, sem0).wait()
        cp_self = pltpu.async_copy(h1_hbm.at[nodesv], selfv, sem_self)

        for blk16 in range(SB // 16):
            qvec = (nodesv[pl.ds(blk16 * 16, 16)] & (QR - 1)) * DEG
            for m in range(16):
                si = blk16 * 16 + m
                start = qvec[m]
                for k in range(DEG // 16):
                    flat = si * DEG + k * 16
                    idxv[flat // CHI, pl.ds(flat % CHI, 16)] = (
                        adjrows[si, pl.ds(start + k * 16, 16)])

        for b, (neigh, sem) in enumerate(bufs):
            pltpu.async_copy(h1_hbm.at[idxv.at[b]], neigh, sem)

        def chunk_group(i, _):
            for b, (neigh, sem) in enumerate(bufs):
                cc = i * NBUF + b
                pltpu.make_async_copy(
                    h1_hbm.at[idxv.at[cc]], neigh, sem).wait()
                _reduce_groups(neigh, CHN, DEG, NV, aggv, cc * CHN * H, H)
                @pl.when(cc + NBUF < NCH)
                def _():
                    pltpu.async_copy(
                        h1_hbm.at[idxv.at[cc + NBUF]], neigh, sem)
            return 0

        lax.fori_loop(0, NCH // NBUF, chunk_group, 0)
        cp_self.wait()
        pltpu.sync_copy(selfv, self_hbm.at[pl.ds(base, SB)])
        pltpu.sync_copy(aggv, agg_hbm.at[pl.ds(base * H, SB * H)])

    kern = pl.kernel(
        body,
        out_type=(
            jax.ShapeDtypeStruct((B, H), jnp.float32),
            jax.ShapeDtypeStruct((B * H,), jnp.float32),
        ),
        mesh=_sc_mesh(),
        scratch_types=[
            pltpu.VMEM((SB,), jnp.int32),
            pltpu.VMEM((SB,), jnp.int32),
            pltpu.VMEM((SB, 128), jnp.int32),
            pltpu.VMEM((NCH, CHI), jnp.int32),
            pltpu.VMEM((SB, H), jnp.float32),
            pltpu.VMEM((CHI, H), jnp.float32),
            pltpu.VMEM((CHI, H), jnp.float32),
            pltpu.VMEM((SB * H,), jnp.float32),
            pltpu.SemaphoreType.DMA,
            pltpu.SemaphoreType.DMA,
            pltpu.SemaphoreType.DMA,
        ],
    )
    return kern


def _layer1_matmul(x, agg, w1aT, w1bT, DEG, blk):
    M, D = x.shape
    H = w1aT.shape[1]
    inv = 1.0 / DEG

    def body(x_ref, g_ref, wa_ref, wb_ref, o_ref):
        h = jnp.dot(x_ref[...], wa_ref[...], preferred_element_type=jnp.float32)
        h = h + jnp.dot(g_ref[...] * inv, wb_ref[...],
                        preferred_element_type=jnp.float32)
        o_ref[...] = jnp.maximum(h, 0.0)

    return pl.pallas_call(
        body,
        grid=(M // blk,),
        in_specs=[
            pl.BlockSpec((blk, D), lambda i: (i, 0)),
            pl.BlockSpec((blk, D), lambda i: (i, 0)),
            pl.BlockSpec((D, H), lambda i: (0, 0)),
            pl.BlockSpec((D, H), lambda i: (0, 0)),
        ],
        out_specs=pl.BlockSpec((blk, H), lambda i: (i, 0)),
        out_shape=jax.ShapeDtypeStruct((M, H), jnp.float32),
    )(x, agg, w1aT, w1bT)


def _layer2_matmul(self_emb, agg2, w2aT, w2bT, cw, DEG):
    B, H = self_emb.shape
    OUT = cw.shape[1]
    inv = 1.0 / DEG

    def body(s_ref, g_ref, wa_ref, wb_ref, cw_ref, o_ref):
        h = jnp.dot(s_ref[...], wa_ref[...], preferred_element_type=jnp.float32)
        h = h + jnp.dot(g_ref[...] * inv, wb_ref[...],
                        preferred_element_type=jnp.float32)
        h = jnp.maximum(h, 0.0)
        o_ref[...] = jnp.dot(h, cw_ref[...], preferred_element_type=jnp.float32)

    return pl.pallas_call(
        body,
        out_shape=jax.ShapeDtypeStruct((B, OUT), jnp.float32),
    )(self_emb, agg2, w2aT, w2bT, cw)


def kernel(nodes, adj, features, w1, w2, cw):
    N, D = features.shape
    DEG = adj.shape[1]
    B = nodes.shape[0]
    H = w1.shape[0]

    CHN = 128 // DEG
    step = _NW * CHN * 4
    NP = ((N + step - 1) // step) * step
    adj_p = jnp.pad(adj, ((0, NP - N), (0, 0)))
    adj3 = adj_p.reshape(_NW, (NP // _NW * DEG) // 128, 128)

    agg_flat = _make_agg_all(N, D, DEG, NP)(adj3, features)
    agg1 = agg_flat.reshape(NP, D)

    w1aT = w1[:, :D].T
    w1bT = w1[:, D:].T
    h1 = _layer1_matmul(features, agg1, w1aT, w1bT, DEG, blk=N // 10)

    adj4 = adj_p.reshape((NP * DEG) // 128, 128)
    self_emb, agg2_flat = _make_seed_agg(N, H, DEG, B)(nodes, adj4, h1)
    agg2 = agg2_flat.reshape(B, H)

    w2aT = w2[:, :H].T
    w2bT = w2[:, H:].T
    return _layer2_matmul(self_emb, agg2, w2aT, w2bT, cw, DEG)
